# baseline (speedup 1.0000x reference)
import functools

import jax
import jax.numpy as jnp
from jax import lax
from jax.experimental import pallas as pl
from jax.experimental.pallas import tpu as pltpu
from jax.experimental.pallas import tpu_sc as plsc

EMB = 64
NC, NS = 2, 16
NW = NC * NS
BLK = 128
LANES = 16
TCB = 32


@functools.cache
def _make_gather(R: int, C: int):
    npair = C // 2
    mesh = plsc.VectorSubcoreMesh(core_axis_name="c", subcore_axis_name="s")

    @functools.partial(
        pl.kernel,
        out_type=jax.ShapeDtypeStruct((npair, R, 2 * EMB), jnp.float32),
        mesh=mesh,
        compiler_params=pltpu.CompilerParams(
            use_tc_tiling_on_sc=False, needs_layout_passes=False),
        scratch_types=[
            pltpu.VMEM((BLK, C), jnp.int32),
            pltpu.VMEM((C, BLK), jnp.int32),
            pltpu.VMEM((2, BLK, EMB), jnp.float32),
            pltpu.VMEM((2, BLK, EMB), jnp.float32),
            pltpu.SemaphoreType.DMA,
            pltpu.SemaphoreType.DMA,
            pltpu.SemaphoreType.DMA,
            pltpu.SemaphoreType.DMA,
        ],
    )
    def gather_kernel(idx_hbm, table_hbm, out_hbm, raw_v, idx_v, buf_a,
                      buf_b, gs_a, gs_b, os_a, os_b):
        wid = lax.axis_index("s") * NC + lax.axis_index("c")
        base = wid * BLK
        pltpu.sync_copy(idx_hbm.at[wid], raw_v)

        lanes = lax.iota(jnp.int32, LANES)
        for d1 in range(C):
            col = jnp.full((LANES,), d1, jnp.int32)
            for cb in range(BLK // LANES):
                v = plsc.load_gather(raw_v, [lanes + cb * LANES, col])
                idx_v[d1, pl.ds(cb * LANES, LANES)] = v

        def start_gathers(k, buf, sem):
            for h in range(2):
                pltpu.async_copy(
                    table_hbm.at[idx_v.at[2 * k + h]], buf.at[h], sem)

        def wait_gathers(k, buf, sem):
            for h in range(2):
                pltpu.make_async_copy(
                    table_hbm.at[idx_v.at[2 * k + h]], buf.at[h], sem).wait()

        def out_descs(k, buf, sem):
            return [
                pltpu.make_async_copy(
                    buf.at[h],
                    out_hbm.at[k, pl.ds(base, BLK), pl.ds(h * EMB, EMB)],
                    sem)
                for h in range(2)
            ]

        def out_start(k, buf, sem):
            for d in out_descs(k, buf, sem):
                d.start()

        def out_wait(k, buf, sem):
            for d in out_descs(k, buf, sem):
                d.wait()

        start_gathers(0, buf_a, gs_a)

        def body(it, _):
            s0 = it * 2
            s1 = s0 + 1
            wait_gathers(s0, buf_a, gs_a)

            @pl.when(it > 0)
            def _():
                out_wait(s1 - 2, buf_b, os_b)

            start_gathers(s1, buf_b, gs_b)
            out_start(s0, buf_a, os_a)
            wait_gathers(s1, buf_b, gs_b)
            out_wait(s0, buf_a, os_a)

            @pl.when(s0 + 2 < npair)
            def _():
                start_gathers(s0 + 2, buf_a, gs_a)

            out_start(s1, buf_b, os_b)
            return ()

        lax.fori_loop(0, (npair - 1) // 2, body, (), unroll=False)
        wait_gathers(npair - 1, buf_a, gs_a)
        out_wait(npair - 2, buf_b, os_b)
        out_start(npair - 1, buf_a, os_a)
        out_wait(npair - 1, buf_a, os_a)

    return gather_kernel


@functools.cache
def _make_format(R: int, C: int):
    npair = C // 2

    def format_kernel(x_ref, o_ref):
        x = x_ref[0]
        for t in range(TCB):
            panel = x[t * BLK:(t + 1) * BLK]
            y = panel.T
            o_ref[:, :, t] = y.reshape(2, EMB // 8, 8, BLK)

    return pl.pallas_call(
        format_kernel,
        grid=(npair, NW // TCB),
        in_specs=[pl.BlockSpec((1, TCB * BLK, 2 * EMB),
                               lambda k, i: (k, i, 0))],
        out_specs=pl.BlockSpec((2, EMB // 8, TCB, 8, BLK),
                               lambda k, i: (k, 0, i, 0, 0)),
        out_shape=jax.ShapeDtypeStruct((C, EMB // 8, NW, 8, BLK),
                                       jnp.float32),
        compiler_params=pltpu.CompilerParams(
            dimension_semantics=("arbitrary", "arbitrary")),
    )


def kernel(multi_hot, table):
    rows, cols = multi_hot.shape
    idx = multi_hot.astype(jnp.int32).reshape(NW, BLK, cols)
    packed = _make_gather(rows, cols)(idx, table)
    out5 = _make_format(rows, cols)(packed)
    return out5.transpose(2, 4, 0, 1, 3).reshape(rows, cols, EMB)


# speedup vs baseline: 1.0603x; 1.0603x over previous
import functools

import jax
import jax.numpy as jnp
from jax import lax
from jax.experimental import pallas as pl
from jax.experimental.pallas import tpu as pltpu
from jax.experimental.pallas import tpu_sc as plsc

EMB = 64
NC, NS = 2, 16
NW = NC * NS
BLK = 128
LANES = 16
TCB = 32


@functools.cache
def _make_gather(R: int, C: int):
    npair = C // 2
    mesh = plsc.VectorSubcoreMesh(core_axis_name="c", subcore_axis_name="s")

    @functools.partial(
        pl.kernel,
        out_type=jax.ShapeDtypeStruct((npair, R, 2 * EMB), jnp.float32),
        mesh=mesh,
        compiler_params=pltpu.CompilerParams(
            use_tc_tiling_on_sc=False, needs_layout_passes=False),
        scratch_types=[
            pltpu.VMEM((BLK, C), jnp.int32),
            pltpu.VMEM((C, BLK), jnp.int32),
            pltpu.VMEM((2, BLK, EMB), jnp.float32),
            pltpu.VMEM((2, BLK, EMB), jnp.float32),
            pltpu.VMEM((2, BLK, EMB), jnp.float32),
            pltpu.VMEM((2, BLK, EMB), jnp.float32),
            pltpu.SemaphoreType.DMA,
            pltpu.SemaphoreType.DMA,
            pltpu.SemaphoreType.DMA,
            pltpu.SemaphoreType.DMA,
            pltpu.SemaphoreType.DMA,
            pltpu.SemaphoreType.DMA,
            pltpu.SemaphoreType.DMA,
            pltpu.SemaphoreType.DMA,
        ],
    )
    def gather_kernel(idx_hbm, table_hbm, out_hbm, raw_v, idx_v, buf_a,
                      buf_b, buf_c, buf_d, gs_a, gs_b, gs_c, gs_d,
                      os_a, os_b, os_c, os_d):
        wid = lax.axis_index("s") * NC + lax.axis_index("c")
        base = wid * BLK
        pltpu.sync_copy(idx_hbm.at[wid], raw_v)

        lanes = lax.iota(jnp.int32, LANES)
        for d1 in range(C):
            col = jnp.full((LANES,), d1, jnp.int32)
            for cb in range(BLK // LANES):
                v = plsc.load_gather(raw_v, [lanes + cb * LANES, col])
                idx_v[d1, pl.ds(cb * LANES, LANES)] = v

        def start_gathers(k, buf, sem):
            for h in range(2):
                pltpu.async_copy(
                    table_hbm.at[idx_v.at[2 * k + h]], buf.at[h], sem)

        def wait_gathers(k, buf, sem):
            for h in range(2):
                pltpu.make_async_copy(
                    table_hbm.at[idx_v.at[2 * k + h]], buf.at[h], sem).wait()

        def out_descs(k, buf, sem):
            return [
                pltpu.make_async_copy(
                    buf.at[h],
                    out_hbm.at[k, pl.ds(base, BLK), pl.ds(h * EMB, EMB)],
                    sem)
                for h in range(2)
            ]

        def out_start(k, buf, sem):
            for d in out_descs(k, buf, sem):
                d.start()

        def out_wait(k, buf, sem):
            for d in out_descs(k, buf, sem):
                d.wait()

        bufs = (buf_a, buf_b, buf_c, buf_d)
        gss = (gs_a, gs_b, gs_c, gs_d)
        oss = (os_a, os_b, os_c, os_d)

        for q in range(3):
            start_gathers(q, bufs[q], gss[q])

        def body(it, _):
            for q in range(4):
                k = it * 4 + q
                jn = (q + 3) % 4
                wait_gathers(k, bufs[q], gss[q])
                out_start(k, bufs[q], oss[q])

                @pl.when(k >= 1)
                def _():
                    out_wait(k - 1, bufs[jn], oss[jn])

                @pl.when(k + 3 < npair)
                def _():
                    start_gathers(k + 3, bufs[jn], gss[jn])
            return ()

        lax.fori_loop(0, (npair - 1) // 4, body, (), unroll=False)
        qt = (npair - 1) % 4
        wait_gathers(npair - 1, bufs[qt], gss[qt])
        out_start(npair - 1, bufs[qt], oss[qt])
        out_wait(npair - 2, bufs[(npair - 2) % 4], oss[(npair - 2) % 4])
        out_wait(npair - 1, bufs[qt], oss[qt])

    return gather_kernel


@functools.cache
def _make_format(R: int, C: int):
    npair = C // 2

    def format_kernel(x_ref, o_ref):
        x = x_ref[0]
        for t in range(TCB):
            panel = x[t * BLK:(t + 1) * BLK]
            y = panel.T
            o_ref[:, :, t] = y.reshape(2, EMB // 8, 8, BLK)

    return pl.pallas_call(
        format_kernel,
        grid=(npair, NW // TCB),
        in_specs=[pl.BlockSpec((1, TCB * BLK, 2 * EMB),
                               lambda k, i: (k, i, 0))],
        out_specs=pl.BlockSpec((2, EMB // 8, TCB, 8, BLK),
                               lambda k, i: (k, 0, i, 0, 0)),
        out_shape=jax.ShapeDtypeStruct((C, EMB // 8, NW, 8, BLK),
                                       jnp.float32),
        compiler_params=pltpu.CompilerParams(
            dimension_semantics=("arbitrary", "arbitrary")),
    )


def kernel(multi_hot, table):
    rows, cols = multi_hot.shape
    idx = multi_hot.astype(jnp.int32).reshape(NW, BLK, cols)
    packed = _make_gather(rows, cols)(idx, table)
    out5 = _make_format(rows, cols)(packed)
    return out5.transpose(2, 4, 0, 1, 3).reshape(rows, cols, EMB)


# speedup vs baseline: 1.0637x; 1.0032x over previous
import functools

import jax
import jax.numpy as jnp
from jax import lax
from jax.experimental import pallas as pl
from jax.experimental.pallas import tpu as pltpu
from jax.experimental.pallas import tpu_sc as plsc

EMB = 64
NC, NS = 2, 16
NW = NC * NS
BLK = 128
LANES = 16
TCB = 32


@functools.cache
def _make_gather(R: int, C: int):
    npair = C // 2
    mesh = plsc.VectorSubcoreMesh(core_axis_name="c", subcore_axis_name="s")

    @functools.partial(
        pl.kernel,
        out_type=jax.ShapeDtypeStruct((npair, R, 2 * EMB), jnp.float32),
        mesh=mesh,
        compiler_params=pltpu.CompilerParams(
            use_tc_tiling_on_sc=False, needs_layout_passes=False),
        scratch_types=[
            pltpu.VMEM((BLK, C), jnp.int32),
            pltpu.VMEM((C, BLK), jnp.int32),
            pltpu.VMEM((2, BLK, EMB), jnp.float32),
            pltpu.VMEM((2, BLK, EMB), jnp.float32),
            pltpu.VMEM((2, BLK, EMB), jnp.float32),
            pltpu.VMEM((2, BLK, EMB), jnp.float32),
            pltpu.VMEM((2, BLK, EMB), jnp.float32),
            pltpu.SemaphoreType.DMA,
            pltpu.SemaphoreType.DMA,
            pltpu.SemaphoreType.DMA,
            pltpu.SemaphoreType.DMA,
            pltpu.SemaphoreType.DMA,
            pltpu.SemaphoreType.DMA,
            pltpu.SemaphoreType.DMA,
            pltpu.SemaphoreType.DMA,
            pltpu.SemaphoreType.DMA,
            pltpu.SemaphoreType.DMA,
        ],
    )
    def gather_kernel(idx_hbm, table_hbm, out_hbm, raw_v, idx_v, buf_a,
                      buf_b, buf_c, buf_d, buf_e, gs_a, gs_b, gs_c, gs_d,
                      gs_e, os_a, os_b, os_c, os_d, os_e):
        wid = lax.axis_index("s") * NC + lax.axis_index("c")
        base = wid * BLK
        pltpu.sync_copy(idx_hbm.at[wid], raw_v)

        lanes = lax.iota(jnp.int32, LANES)
        for d1 in range(C):
            col = jnp.full((LANES,), d1, jnp.int32)
            for cb in range(BLK // LANES):
                v = plsc.load_gather(raw_v, [lanes + cb * LANES, col])
                idx_v[d1, pl.ds(cb * LANES, LANES)] = v

        def start_gathers(k, buf, sem):
            for h in range(2):
                pltpu.async_copy(
                    table_hbm.at[idx_v.at[2 * k + h]], buf.at[h], sem)

        def wait_gathers(k, buf, sem):
            for h in range(2):
                pltpu.make_async_copy(
                    table_hbm.at[idx_v.at[2 * k + h]], buf.at[h], sem).wait()

        def out_descs(k, buf, sem):
            return [
                pltpu.make_async_copy(
                    buf.at[h],
                    out_hbm.at[k, pl.ds(base, BLK), pl.ds(h * EMB, EMB)],
                    sem)
                for h in range(2)
            ]

        def out_start(k, buf, sem):
            for d in out_descs(k, buf, sem):
                d.start()

        def out_wait(k, buf, sem):
            for d in out_descs(k, buf, sem):
                d.wait()

        nbuf = 5
        bufs = (buf_a, buf_b, buf_c, buf_d, buf_e)
        gss = (gs_a, gs_b, gs_c, gs_d, gs_e)
        oss = (os_a, os_b, os_c, os_d, os_e)

        for q in range(nbuf - 1):
            start_gathers(q, bufs[q], gss[q])

        def body(it, _):
            for q in range(nbuf):
                k = it * nbuf + q
                jn = (q + nbuf - 1) % nbuf
                wait_gathers(k, bufs[q], gss[q])
                out_start(k, bufs[q], oss[q])

                @pl.when(k >= 1)
                def _():
                    out_wait(k - 1, bufs[jn], oss[jn])

                @pl.when(k + nbuf - 1 < npair)
                def _():
                    start_gathers(k + nbuf - 1, bufs[jn], gss[jn])
            return ()

        lax.fori_loop(0, npair // nbuf, body, (), unroll=False)
        out_wait(npair - 1, bufs[(npair - 1) % nbuf], oss[(npair - 1) % nbuf])

    return gather_kernel


@functools.cache
def _make_format(R: int, C: int):
    npair = C // 2

    def format_kernel(x_ref, o_ref):
        x = x_ref[0]
        for t in range(TCB):
            panel = x[t * BLK:(t + 1) * BLK]
            y = panel.T
            o_ref[:, :, t] = y.reshape(2, EMB // 8, 8, BLK)

    return pl.pallas_call(
        format_kernel,
        grid=(npair, NW // TCB),
        in_specs=[pl.BlockSpec((1, TCB * BLK, 2 * EMB),
                               lambda k, i: (k, i, 0))],
        out_specs=pl.BlockSpec((2, EMB // 8, TCB, 8, BLK),
                               lambda k, i: (k, 0, i, 0, 0)),
        out_shape=jax.ShapeDtypeStruct((C, EMB // 8, NW, 8, BLK),
                                       jnp.float32),
        compiler_params=pltpu.CompilerParams(
            dimension_semantics=("arbitrary", "arbitrary")),
    )


def kernel(multi_hot, table):
    rows, cols = multi_hot.shape
    idx = multi_hot.astype(jnp.int32).reshape(NW, BLK, cols)
    packed = _make_gather(rows, cols)(idx, table)
    out5 = _make_format(rows, cols)(packed)
    return out5.transpose(2, 4, 0, 1, 3).reshape(rows, cols, EMB)
